# Initial kernel scaffold; baseline (speedup 1.0000x reference)
import jax
import jax.numpy as jnp
from jax.experimental import pallas as pl


def kernel(x, W, b):
    raise NotImplementedError("write your pallas kernel here")


# speedup vs baseline: 2.6984x; 2.6984x over previous
import jax
import jax.numpy as jnp
from jax import lax
from jax.experimental import pallas as pl
from jax.experimental.pallas import tpu as pltpu

_TOKENS = 8192
_DIM = 2048
_NE = 16
_EPS = 1e-06
_CAP = float(_TOKENS)
_TILE = 512
_GRID = _TOKENS // _TILE


def _body(x_ref, w_ref, b_ref, out_ref, acc_ref):
    i = pl.program_id(0)
    logits = lax.dot_general(
        x_ref[...], w_ref[...], (((1,), (1,)), ((), ())),
        preferred_element_type=jnp.float32,
    ) + b_ref[...]
    m = jnp.max(logits, axis=1, keepdims=True)
    e = jnp.exp(logits - m)
    p = e / jnp.sum(e, axis=1, keepdims=True)
    pmax = jnp.max(p, axis=1, keepdims=True)
    idx = lax.broadcasted_iota(jnp.int32, p.shape, 1)
    first = jnp.min(jnp.where(p == pmax, idx, _NE), axis=1, keepdims=True)
    masked = jnp.where(idx == first, p, 0.0)

    @pl.when(i == 0)
    def _():
        acc_ref[...] = jnp.zeros_like(acc_ref)

    acc_ref[...] += jnp.sum(masked, axis=0, keepdims=True)
    out_ref[pl.ds(i * _TILE, _TILE), :] = masked

    @pl.when(i == _GRID - 1)
    def _():
        scale = _CAP / (acc_ref[...] + _EPS)
        out_ref[...] = out_ref[...] * scale


def kernel(x, W, b):
    b2 = b.reshape(1, _NE)
    return pl.pallas_call(
        _body,
        grid=(_GRID,),
        in_specs=[
            pl.BlockSpec((_TILE, _DIM), lambda i: (i, 0)),
            pl.BlockSpec((_NE, _DIM), lambda i: (0, 0)),
            pl.BlockSpec((1, _NE), lambda i: (0, 0)),
        ],
        out_specs=pl.BlockSpec((_TOKENS, _NE), lambda i: (0, 0)),
        out_shape=jax.ShapeDtypeStruct((_TOKENS, _NE), jnp.float32),
        scratch_shapes=[pltpu.VMEM((1, _NE), jnp.float32)],
    )(x, W, b2)
